# baseline (speedup 1.0000x reference)
import functools

import jax
import jax.numpy as jnp
from jax import lax
from jax.experimental import pallas as pl
from jax.experimental.pallas import tpu as pltpu
from jax.experimental.pallas import tpu_sc as plsc

B = 16384
L = 20
NEG = 20
D = 64

NC, NS = 2, 16
NW = NC * NS
CH = 128
NCH = 2
BC = B // NCH


def _sc_gather(table_a, idx_c, neg_pad, table_b, ctx_pad, b_off):
    gr = BC // NW
    kg = gr // CH
    kh = kg // 2
    mesh = plsc.VectorSubcoreMesh(
        core_axis_name="c", subcore_axis_name="s",
        num_cores=NC, num_subcores=NS)

    @functools.partial(
        pl.kernel,
        out_type=(jax.ShapeDtypeStruct((BC // 2, 2 * D), jnp.float32),
                  jax.ShapeDtypeStruct((NEG, BC // 2, 2 * D), jnp.float32),
                  jax.ShapeDtypeStruct((L, BC // 2, 2 * D), jnp.float32)),
        mesh=mesh,
        scratch_types=[
            pltpu.VMEM((2, kh, CH), jnp.int32),
            pltpu.VMEM((2, gr // 2, D), jnp.float32),
            pltpu.VMEM((gr, 24), jnp.int32),
            pltpu.VMEM((gr,), jnp.int32),
            pltpu.SemaphoreType.DMA,
        ],
        compiler_params=pltpu.CompilerParams(use_tc_tiling_on_sc=False,
                                             needs_layout_passes=False),
    )
    def k(ta, ic, inn, tb, ixx, oc, on, ox, idx_v, rows_v, tile_v, tile_c, sem):
        wid = lax.axis_index("s") * NC + lax.axis_index("c")
        b0 = wid * gr
        h0 = wid * (gr // 2)
        lanes = lax.iota(jnp.int32, 16)

        def fill_idx(n):
            col = jnp.zeros((16,), jnp.int32) + n
            for par in range(2):
                for q in range(gr // 2 // 16):
                    row = 2 * (lanes + q * 16) + par
                    v = plsc.load_gather(tile_v, [row, col])
                    idx_v[par, q // 8, pl.ds((q % 8) * 16, 16)] = v

        def gather_group(tbl):
            cops = [pltpu.async_copy(tbl.at[idx_v.at[par, j]],
                                     rows_v.at[par, pl.ds(j * CH, CH)], sem)
                    for par in range(2) for j in range(kh)]
            for c in cops:
                c.wait()

        def scatter(dst):
            for par in range(2):
                pltpu.sync_copy(rows_v.at[par],
                                dst.at[pl.ds(h0, gr // 2),
                                       pl.ds(par * D, D)])

        pltpu.sync_copy(ic.at[pl.ds(b_off + b0, gr)], tile_c)
        for par in range(2):
            for q in range(gr // 2 // 16):
                v = plsc.load_gather(tile_c, [2 * (lanes + q * 16) + par])
                idx_v[par, q // 8, pl.ds((q % 8) * 16, 16)] = v
        gather_group(ta)
        scatter(oc)

        def do_stream(tbl, ihbm, ohbm):
            pltpu.sync_copy(ihbm.at[pl.ds(b_off + b0, gr), pl.ds(0, 24)],
                            tile_v)

            def body(n, carry):
                fill_idx(n)
                gather_group(tbl)
                scatter(ohbm.at[n])
                return carry

            lax.fori_loop(0, NEG, body, 0)

        do_stream(ta, inn, on)
        do_stream(tb, ixx, ox)

    return k(table_a, idx_c, neg_pad, table_b, ctx_pad)


TB = 512
TH = TB // 2
NT = BC // TB


def _logsig(x):
    return jnp.minimum(x, 0.0) - jnp.log1p(jnp.exp(-jnp.abs(x)))


def _tc_partial(center_rows, neg_rows, ctx_rows, cl, xl):
    def body(cen_ref, neg_ref, ctx_ref, cl_ref, xl_ref, out_ref, acc_ref):
        i = pl.program_id(0)
        m = lax.dot_general(cl_ref[...], xl_ref[...],
                            (((0,), (0,)), ((), ())),
                            preferred_element_type=jnp.float32)
        z = jnp.zeros((D, D), jnp.float32)
        m2 = jnp.concatenate(
            [jnp.concatenate([m, z], axis=1),
             jnp.concatenate([z, m], axis=1)], axis=0)
        cen2 = cen_ref[...]
        neg2 = neg_ref[...].reshape(NEG * TH, 2 * D)
        ctx2 = ctx_ref[...].reshape(L * TH, 2 * D)
        amc = lax.dot_general(cen2, m2, (((1,), (0,)), ((), ())),
                              preferred_element_type=jnp.float32)
        amn = lax.dot_general(neg2, -m2, (((1,), (0,)), ((), ())),
                              preferred_element_type=jnp.float32)
        act = amc.astype(jnp.bfloat16).T
        ant = amn.astype(jnp.bfloat16).T
        ct = ctx2.astype(jnp.bfloat16).T
        rows = []
        for n in range(1 + NEG):
            a_n = act if n == 0 else ant[:, (n - 1) * TH:n * TH]
            for l in range(L):
                p = a_n * ct[:, l * TH:(l + 1) * TH]
                rows.append(jnp.sum(p[:D], axis=0,
                                    dtype=jnp.bfloat16))
                rows.append(jnp.sum(p[D:], axis=0,
                                    dtype=jnp.bfloat16))
        s_all = jnp.stack(rows).astype(jnp.float32)
        acc = jnp.sum(_logsig(s_all))

        @pl.when(i == 0)
        def _():
            acc_ref[0, 0] = acc

        @pl.when(i > 0)
        def _():
            acc_ref[0, 0] += acc

        @pl.when(i == NT - 1)
        def _():
            out_ref[0, 0] = acc_ref[0, 0]

    res = pl.pallas_call(
        body,
        grid=(NT,),
        in_specs=[
            pl.BlockSpec((TH, 2 * D), lambda i: (i, 0)),
            pl.BlockSpec((NEG, TH, 2 * D), lambda i: (0, i, 0)),
            pl.BlockSpec((L, TH, 2 * D), lambda i: (0, i, 0)),
            pl.BlockSpec((128, D), lambda i: (0, 0)),
            pl.BlockSpec((128, D), lambda i: (0, 0)),
        ],
        out_specs=pl.BlockSpec(memory_space=pltpu.SMEM),
        out_shape=jax.ShapeDtypeStruct((1, 1), jnp.float32),
        scratch_shapes=[pltpu.SMEM((1, 1), jnp.float32)],
    )(center_rows, neg_rows, ctx_rows, cl, xl)
    return res[0, 0]


def kernel(center_embedding, center_linear, context_embedding, context_linear,
           center, contexts, negatives):
    idx_c = center.astype(jnp.int32)
    pad = ((0, 0), (0, 128 - NEG))
    neg_pad = jnp.pad(negatives.astype(jnp.int32), pad)
    ctx_pad = jnp.pad(contexts.astype(jnp.int32), pad)

    parts = []
    for c in range(NCH):
        center_rows, neg_rows, ctx_rows = _sc_gather(
            center_embedding, idx_c, neg_pad,
            context_embedding, ctx_pad, c * BC)
        parts.append(_tc_partial(center_rows, neg_rows, ctx_rows,
                                 center_linear, context_linear))

    total = parts[0]
    for p in parts[1:]:
        total = total + p
    return -total / float(B * (1 + NEG) * L)


# speedup vs baseline: 1.0920x; 1.0920x over previous
import functools

import jax
import jax.numpy as jnp
from jax import lax
from jax.experimental import pallas as pl
from jax.experimental.pallas import tpu as pltpu
from jax.experimental.pallas import tpu_sc as plsc

B = 16384
L = 20
NEG = 20
D = 64

NC, NS = 2, 16
NW = NC * NS
CH = 128
NCH = 2
BC = B // NCH

GRC = BC // NW
KHC = GRC // 2 // CH

_MESH = plsc.VectorSubcoreMesh(
    core_axis_name="c", subcore_axis_name="s",
    num_cores=NC, num_subcores=NS)
_SC_PARAMS = pltpu.CompilerParams(use_tc_tiling_on_sc=False,
                                  needs_layout_passes=False)
_SC_SCRATCH = [
    pltpu.VMEM((2, KHC, CH), jnp.int32),
    pltpu.VMEM((2, GRC // 2, D), jnp.float32),
    pltpu.VMEM((GRC * NEG,), jnp.int32),
    pltpu.SemaphoreType.DMA,
]

_LANES16 = lambda: lax.iota(jnp.int32, 16)


def _worker_ctx():
    wid = lax.axis_index("s") * NC + lax.axis_index("c")
    return wid * GRC, wid * (GRC // 2)


def _fill_idx_strided(tile_v, idx_v, n):
    lanes = _LANES16()
    for par in range(2):
        for q in range(GRC // 2 // 16):
            flat = (2 * (lanes + q * 16) + par) * NEG + n
            v = plsc.load_gather(tile_v, [flat])
            idx_v[par, q // 8, pl.ds((q % 8) * 16, 16)] = v


def _gather_group(tbl, idx_v, rows_v, sem):
    cops = [pltpu.async_copy(tbl.at[idx_v.at[par, j]],
                             rows_v.at[par, pl.ds(j * CH, CH)], sem)
            for par in range(2) for j in range(KHC)]
    for c in cops:
        c.wait()


def _scatter(rows_v, dst, h0):
    for par in range(2):
        pltpu.sync_copy(rows_v.at[par],
                        dst.at[pl.ds(h0, GRC // 2), pl.ds(par * D, D)])


def _sc_gather_a(table_a, idx_c, neg_flat):
    @functools.partial(
        pl.kernel,
        out_type=(jax.ShapeDtypeStruct((BC // 2, 2 * D), jnp.float32),
                  jax.ShapeDtypeStruct((NEG, BC // 2, 2 * D), jnp.float32)),
        mesh=_MESH,
        scratch_types=_SC_SCRATCH,
        compiler_params=_SC_PARAMS,
    )
    def k(ta, ic, inn, oc, on, idx_v, rows_v, tile_v, sem):
        b0, h0 = _worker_ctx()
        lanes = _LANES16()

        pltpu.sync_copy(ic.at[pl.ds(b0, GRC)], tile_v.at[pl.ds(0, GRC)])
        for par in range(2):
            for q in range(GRC // 2 // 16):
                v = plsc.load_gather(tile_v, [2 * (lanes + q * 16) + par])
                idx_v[par, q // 8, pl.ds((q % 8) * 16, 16)] = v
        _gather_group(ta, idx_v, rows_v, sem)
        _scatter(rows_v, oc, h0)

        pltpu.sync_copy(inn.at[pl.ds(b0 * NEG, GRC * NEG)], tile_v)

        def body(n, carry):
            _fill_idx_strided(tile_v, idx_v, n)
            _gather_group(ta, idx_v, rows_v, sem)
            _scatter(rows_v, on.at[n], h0)
            return carry

        lax.fori_loop(0, NEG, body, 0)

    return k(table_a, idx_c, neg_flat)


def _sc_gather_b(table_b, ctx_flat):
    @functools.partial(
        pl.kernel,
        out_type=jax.ShapeDtypeStruct((L, BC // 2, 2 * D), jnp.float32),
        mesh=_MESH,
        scratch_types=_SC_SCRATCH,
        compiler_params=_SC_PARAMS,
    )
    def k(tb, ixx, ox, idx_v, rows_v, tile_v, sem):
        b0, h0 = _worker_ctx()
        pltpu.sync_copy(ixx.at[pl.ds(b0 * NEG, GRC * NEG)], tile_v)

        def body(n, carry):
            _fill_idx_strided(tile_v, idx_v, n)
            _gather_group(tb, idx_v, rows_v, sem)
            _scatter(rows_v, ox.at[n], h0)
            return carry

        lax.fori_loop(0, L, body, 0)

    return k(table_b, ctx_flat)


TB = 512
TH = TB // 2
NT = BC // TB


def _logsig(x):
    return jnp.minimum(x, 0.0) - jnp.log1p(jnp.exp(-jnp.abs(x)))


def _tc_partial(center_rows, neg_rows, ctx_rows, cl, xl):
    def body(cen_ref, neg_ref, ctx_ref, cl_ref, xl_ref, out_ref, acc_ref):
        i = pl.program_id(0)
        m = lax.dot_general(cl_ref[...], xl_ref[...],
                            (((0,), (0,)), ((), ())),
                            preferred_element_type=jnp.float32)
        z = jnp.zeros((D, D), jnp.float32)
        m2 = jnp.concatenate(
            [jnp.concatenate([m, z], axis=1),
             jnp.concatenate([z, m], axis=1)], axis=0)
        cen2 = cen_ref[...]
        neg2 = neg_ref[...].reshape(NEG * TH, 2 * D)
        ctx2 = ctx_ref[...].reshape(L * TH, 2 * D)
        amc = lax.dot_general(cen2, m2, (((1,), (0,)), ((), ())),
                              preferred_element_type=jnp.float32)
        amn = lax.dot_general(neg2, -m2, (((1,), (0,)), ((), ())),
                              preferred_element_type=jnp.float32)
        act = amc.astype(jnp.bfloat16).T
        ant = amn.astype(jnp.bfloat16).T
        ct = ctx2.astype(jnp.bfloat16).T
        rows = []
        for n in range(1 + NEG):
            a_n = act if n == 0 else ant[:, (n - 1) * TH:n * TH]
            for l in range(L):
                p = a_n * ct[:, l * TH:(l + 1) * TH]
                rows.append(jnp.sum(p[:D], axis=0,
                                    dtype=jnp.bfloat16))
                rows.append(jnp.sum(p[D:], axis=0,
                                    dtype=jnp.bfloat16))
        s_all = jnp.stack(rows).astype(jnp.float32)
        acc = jnp.sum(_logsig(s_all))

        @pl.when(i == 0)
        def _():
            acc_ref[0, 0] = acc

        @pl.when(i > 0)
        def _():
            acc_ref[0, 0] += acc

        @pl.when(i == NT - 1)
        def _():
            out_ref[0, 0] = acc_ref[0, 0]

    res = pl.pallas_call(
        body,
        grid=(NT,),
        in_specs=[
            pl.BlockSpec((TH, 2 * D), lambda i: (i, 0)),
            pl.BlockSpec((NEG, TH, 2 * D), lambda i: (0, i, 0)),
            pl.BlockSpec((L, TH, 2 * D), lambda i: (0, i, 0)),
            pl.BlockSpec((128, D), lambda i: (0, 0)),
            pl.BlockSpec((128, D), lambda i: (0, 0)),
        ],
        out_specs=pl.BlockSpec(memory_space=pltpu.SMEM),
        out_shape=jax.ShapeDtypeStruct((1, 1), jnp.float32),
        scratch_shapes=[pltpu.SMEM((1, 1), jnp.float32)],
    )(center_rows, neg_rows, ctx_rows, cl, xl)
    return res[0, 0]


def kernel(center_embedding, center_linear, context_embedding, context_linear,
           center, contexts, negatives):
    parts = []
    for c in range(NCH):
        sl = slice(c * BC, (c + 1) * BC)
        center_rows, neg_rows = _sc_gather_a(
            center_embedding, center[sl].astype(jnp.int32),
            negatives[sl].astype(jnp.int32).reshape(-1))
        ctx_rows = _sc_gather_b(
            context_embedding, contexts[sl].astype(jnp.int32).reshape(-1))
        parts.append(_tc_partial(center_rows, neg_rows, ctx_rows,
                                 center_linear, context_linear))

    total = parts[0]
    for p in parts[1:]:
        total = total + p
    return -total / float(B * (1 + NEG) * L)


# speedup vs baseline: 1.1203x; 1.0259x over previous
import functools

import jax
import jax.numpy as jnp
from jax import lax
from jax.experimental import pallas as pl
from jax.experimental.pallas import tpu as pltpu
from jax.experimental.pallas import tpu_sc as plsc

B = 16384
L = 20
NEG = 20
D = 64

NC, NS = 2, 16
NW = NC * NS
CH = 128
NCH = 2
BC = B // NCH

GRC = BC // NW
KHC = GRC // 2 // CH

_MESH = plsc.VectorSubcoreMesh(
    core_axis_name="c", subcore_axis_name="s",
    num_cores=NC, num_subcores=NS)
_SC_PARAMS = pltpu.CompilerParams(use_tc_tiling_on_sc=False,
                                  needs_layout_passes=False)
_SC_SCRATCH = [
    pltpu.VMEM((2, 2, KHC, CH), jnp.int32),
    pltpu.VMEM((2, 2, GRC // 2, D), jnp.float32),
    pltpu.VMEM((GRC * NEG,), jnp.int32),
    pltpu.SemaphoreType.DMA,
    pltpu.SemaphoreType.DMA,
]

_LANES16 = lambda: lax.iota(jnp.int32, 16)


def _worker_ctx():
    wid = lax.axis_index("s") * NC + lax.axis_index("c")
    return wid * GRC, wid * (GRC // 2)


def _fill_idx_strided(tile_v, idx_v, n):
    lanes = _LANES16()
    for par in range(2):
        for q in range(GRC // 2 // 16):
            flat = (2 * (lanes + q * 16) + par) * NEG + n
            v = plsc.load_gather(tile_v, [flat])
            idx_v[par, q // 8, pl.ds((q % 8) * 16, 16)] = v


def _run_stream(groups, tbl, idx_v, rows_v, sem, sem2, h0):
    pend_gather = []
    pend_scatter = []

    def issue(g, fill, dst):
        buf = g % 2
        fill(idx_v.at[buf])
        pend_gather.extend(
            pltpu.async_copy(tbl.at[idx_v.at[buf, par, j]],
                             rows_v.at[buf, par, pl.ds(j * CH, CH)], sem)
            for par in range(2) for j in range(KHC))
        for c in pend_gather:
            c.wait()
        pend_gather.clear()
        if pend_scatter:
            for c in pend_scatter:
                c.wait()
            pend_scatter.clear()
        pend_scatter.extend(
            pltpu.async_copy(rows_v.at[buf, par],
                             dst.at[pl.ds(h0, GRC // 2),
                                    pl.ds(par * D, D)], sem2)
            for par in range(2))

    for g, (fill, dst) in enumerate(groups):
        issue(g, fill, dst)
    for c in pend_scatter:
        c.wait()


def _fill_from_tile(tile_v, n):
    def fill(idx_b):
        _fill_idx_strided(tile_v, idx_b, n)
    return fill


def _sc_gather_a(table_a, idx_c, neg_flat):
    @functools.partial(
        pl.kernel,
        out_type=(jax.ShapeDtypeStruct((BC // 2, 2 * D), jnp.float32),
                  jax.ShapeDtypeStruct((NEG, BC // 2, 2 * D), jnp.float32)),
        mesh=_MESH,
        scratch_types=_SC_SCRATCH,
        compiler_params=_SC_PARAMS,
    )
    def k(ta, ic, inn, oc, on, idx_v, rows_v, tile_v, sem, sem2):
        b0, h0 = _worker_ctx()
        lanes = _LANES16()

        pltpu.sync_copy(ic.at[pl.ds(b0, GRC)], tile_v.at[pl.ds(0, GRC)])

        def fill_center(idx_b):
            for par in range(2):
                for q in range(GRC // 2 // 16):
                    v = plsc.load_gather(tile_v, [2 * (lanes + q * 16) + par])
                    idx_b[par, q // 8, pl.ds((q % 8) * 16, 16)] = v

        groups = [(fill_center, oc)]
        _run_stream(groups, ta, idx_v, rows_v, sem, sem2, h0)

        pltpu.sync_copy(inn.at[pl.ds(b0 * NEG, GRC * NEG)], tile_v)
        groups = [(_fill_from_tile(tile_v, n), on.at[n]) for n in range(NEG)]
        _run_stream(groups, ta, idx_v, rows_v, sem, sem2, h0)

    return k(table_a, idx_c, neg_flat)


def _sc_gather_b(table_b, ctx_flat):
    @functools.partial(
        pl.kernel,
        out_type=jax.ShapeDtypeStruct((L, BC // 2, 2 * D), jnp.float32),
        mesh=_MESH,
        scratch_types=_SC_SCRATCH,
        compiler_params=_SC_PARAMS,
    )
    def k(tb, ixx, ox, idx_v, rows_v, tile_v, sem, sem2):
        b0, h0 = _worker_ctx()
        pltpu.sync_copy(ixx.at[pl.ds(b0 * NEG, GRC * NEG)], tile_v)
        groups = [(_fill_from_tile(tile_v, n), ox.at[n]) for n in range(L)]
        _run_stream(groups, tb, idx_v, rows_v, sem, sem2, h0)

    return k(table_b, ctx_flat)


TB = 512
TH = TB // 2
NT = BC // TB


def _logsig(x):
    return jnp.minimum(x, 0.0) - jnp.log1p(jnp.exp(-jnp.abs(x)))


def _tc_partial(center_rows, neg_rows, ctx_rows, cl, xl):
    def body(cen_ref, neg_ref, ctx_ref, cl_ref, xl_ref, out_ref, acc_ref):
        i = pl.program_id(0)
        m = lax.dot_general(cl_ref[...], xl_ref[...],
                            (((0,), (0,)), ((), ())),
                            preferred_element_type=jnp.float32)
        z = jnp.zeros((D, D), jnp.float32)
        m2 = jnp.concatenate(
            [jnp.concatenate([m, z], axis=1),
             jnp.concatenate([z, m], axis=1)], axis=0)
        cen2 = cen_ref[...]
        neg2 = neg_ref[...].reshape(NEG * TH, 2 * D)
        ctx2 = ctx_ref[...].reshape(L * TH, 2 * D)
        amc = lax.dot_general(cen2, m2, (((1,), (0,)), ((), ())),
                              preferred_element_type=jnp.float32)
        amn = lax.dot_general(neg2, -m2, (((1,), (0,)), ((), ())),
                              preferred_element_type=jnp.float32)
        act = amc.astype(jnp.bfloat16).T
        ant = amn.astype(jnp.bfloat16).T
        ct = ctx2.astype(jnp.bfloat16).T
        rows = []
        for n in range(1 + NEG):
            a_n = act if n == 0 else ant[:, (n - 1) * TH:n * TH]
            for l in range(L):
                p = a_n * ct[:, l * TH:(l + 1) * TH]
                rows.append(jnp.sum(p[:D], axis=0,
                                    dtype=jnp.bfloat16))
                rows.append(jnp.sum(p[D:], axis=0,
                                    dtype=jnp.bfloat16))
        s_all = jnp.stack(rows).astype(jnp.float32)
        acc = jnp.sum(_logsig(s_all))

        @pl.when(i == 0)
        def _():
            acc_ref[0, 0] = acc

        @pl.when(i > 0)
        def _():
            acc_ref[0, 0] += acc

        @pl.when(i == NT - 1)
        def _():
            out_ref[0, 0] = acc_ref[0, 0]

    res = pl.pallas_call(
        body,
        grid=(NT,),
        in_specs=[
            pl.BlockSpec((TH, 2 * D), lambda i: (i, 0)),
            pl.BlockSpec((NEG, TH, 2 * D), lambda i: (0, i, 0)),
            pl.BlockSpec((L, TH, 2 * D), lambda i: (0, i, 0)),
            pl.BlockSpec((128, D), lambda i: (0, 0)),
            pl.BlockSpec((128, D), lambda i: (0, 0)),
        ],
        out_specs=pl.BlockSpec(memory_space=pltpu.SMEM),
        out_shape=jax.ShapeDtypeStruct((1, 1), jnp.float32),
        scratch_shapes=[pltpu.SMEM((1, 1), jnp.float32)],
    )(center_rows, neg_rows, ctx_rows, cl, xl)
    return res[0, 0]


def kernel(center_embedding, center_linear, context_embedding, context_linear,
           center, contexts, negatives):
    parts = []
    for c in range(NCH):
        sl = slice(c * BC, (c + 1) * BC)
        center_rows, neg_rows = _sc_gather_a(
            center_embedding, center[sl].astype(jnp.int32),
            negatives[sl].astype(jnp.int32).reshape(-1))
        ctx_rows = _sc_gather_b(
            context_embedding, contexts[sl].astype(jnp.int32).reshape(-1))
        parts.append(_tc_partial(center_rows, neg_rows, ctx_rows,
                                 center_linear, context_linear))

    total = parts[0]
    for p in parts[1:]:
        total = total + p
    return -total / float(B * (1 + NEG) * L)
